# baseline (speedup 1.0000x reference)
import functools

import jax
import jax.numpy as jnp
from jax import lax
from jax.experimental import pallas as pl
from jax.experimental.pallas import tpu as pltpu
from jax.experimental.pallas import tpu_sc as plsc

NUM_CORES = 2
NUM_SUBCORES = 16
NUM_WORKERS = NUM_CORES * NUM_SUBCORES
LANES = 16

B = 4096 * 200
D = 64
G = 128
ROWS_PER_WORKER = B // NUM_WORKERS
CHUNK = 512
GPC = CHUNK // G
NCHUNKS = ROWS_PER_WORKER // CHUNK
OUT_SCALE = 8.0


def _emb_body(ids_hbm, table_hbm, out_hbm, idx_v, rows_v, sem):
    wid = lax.axis_index("s") * NUM_CORES + lax.axis_index("c")
    grp_base = wid * (ROWS_PER_WORKER // G)
    row_base = wid * ROWS_PER_WORKER

    def chunk_body(c, carry):
        g0 = grp_base + c * GPC
        r0 = row_base + c * CHUNK
        pltpu.sync_copy(ids_hbm.at[pl.ds(g0, GPC)], idx_v)
        copies = []
        for j in range(GPC):
            copies.append(
                pltpu.async_copy(
                    table_hbm.at[idx_v.at[j]],
                    rows_v.at[pl.ds(j * G, G)],
                    sem,
                )
            )
        for cp in copies:
            cp.wait()

        def scale_row(r, carry2):
            for cc in range(D // LANES):
                sl = (r, pl.ds(cc * LANES, LANES))
                rows_v[sl] = rows_v[sl] * OUT_SCALE
            return carry2

        lax.fori_loop(0, CHUNK, scale_row, 0, unroll=2)

        pltpu.sync_copy(rows_v, out_hbm.at[pl.ds(r0, CHUNK)])
        return carry

    lax.fori_loop(0, NCHUNKS, chunk_body, 0)


@jax.jit
def _emb(ids_grouped, weight):
    mesh = plsc.VectorSubcoreMesh(
        core_axis_name="c", subcore_axis_name="s",
        num_cores=NUM_CORES, num_subcores=NUM_SUBCORES,
    )
    return pl.kernel(
        _emb_body,
        out_type=jax.ShapeDtypeStruct((B, D), jnp.float32),
        mesh=mesh,
        scratch_types=[
            pltpu.VMEM((GPC, G), jnp.int32),
            pltpu.VMEM((CHUNK, D), jnp.float32),
            pltpu.SemaphoreType.DMA,
        ],
        compiler_params=pltpu.CompilerParams(use_tc_tiling_on_sc=False),
    )(ids_grouped, weight)


def kernel(input_ids, weight):
    ids = input_ids.astype(jnp.int32).reshape(B // G, G)
    out = _emb(ids, weight)
    return out.reshape(input_ids.shape + (D,))
